# baseline (speedup 1.0000x reference)
import functools

import jax
import jax.numpy as jnp
from jax import lax
from jax.experimental import pallas as pl
from jax.experimental.pallas import tpu as pltpu
from jax.experimental.pallas import tpu_sc as plsc

N = 10000
E = 320000
D = 128
H = 128
C = 40

NC = 2
NS = 16
NW = NC * NS
NPAD = 10240
RANGE = NPAD // NW
Q = 640
NQ = E // Q
QV2 = Q // 32
G = 32
GSHIFT = 5
MFLUSH = 1280
MCAP = 2048
NROWS = MCAP // G
PADV = 3

_f32 = jnp.float32
_i32 = jnp.int32


def _sc_rel(ep_h, table_h, out_h, deg_out_h, mode,
            acc, dega, ebufs, fsrc2, fdst, fw, rowss, esems, gsems,
            wid):
    zf = jnp.zeros((16,), _f32)
    zi = jnp.zeros((16,), _i32)
    onef = jnp.ones((16,), _f32)
    rangev = jnp.full((16,), RANGE, _i32)

    lo = wid * RANGE
    lo_v = jnp.full((16,), lo, _i32)
    hi_v = jnp.full((16,), lo + RANGE, _i32)
    lane = lax.iota(_i32, 16)
    trash = jnp.full((16,), MCAP - 16, _i32) + lane
    one_i = jnp.ones((16,), _i32)
    gm1 = jnp.full((16,), G - 1, _i32)

    def zero_row(r, _):
        for j in range(8):
            acc[r, pl.ds(j * 16, 16)] = zf
        if mode == "add":
            dega[r, pl.ds(0, 16)] = zf
        return 0

    lax.fori_loop(0, RANGE + 1, zero_row, 0)

    def fire_e(k, s):
        pltpu.async_copy(ep_h.at[:, pl.ds(k * Q, Q)], ebufs[s], esems[s])

    def wait_e(s):
        pltpu.make_async_copy(ep_h.at[:, pl.ds(0, Q)], ebufs[s], esems[s]).wait()

    def scat3(pos, sv, dv, wv):
        plsc.store_scatter(fsrc2, [lax.shift_right_logical(pos, GSHIFT),
                                   pos & gm1], sv)
        plsc.store_scatter(fdst, [pos], dv)
        plsc.store_scatter(fw, [pos], wv)

    def filter_chunk(s, staged):
        eb = ebufs[s]

        def filt(i, cnt):
            off = i * 32
            cntv = jnp.full((16,), cnt, _i32)
            d_a = eb[1, pl.ds(off, 16)]
            s_a = eb[0, pl.ds(off, 16)]
            w_a = eb[2, pl.ds(off, 16)]
            d_b = eb[1, pl.ds(off + 16, 16)]
            s_b = eb[0, pl.ds(off + 16, 16)]
            w_b = eb[2, pl.ds(off + 16, 16)]
            m_a = (d_a >= lo_v) & (d_a < hi_v)
            m_b = (d_b >= lo_v) & (d_b < hi_v)
            inc_a = plsc.cumsum(m_a.astype(_i32))
            inc_b = plsc.cumsum(m_b.astype(_i32))
            pc_a = plsc.all_reduce_population_count(m_a)
            pc_b = plsc.all_reduce_population_count(m_b)
            pos_a = jnp.where(m_a, cntv + inc_a - one_i, trash)
            pos_b = jnp.where(m_b, cntv + pc_a + inc_b - one_i, trash)
            scat3(pos_a, s_a, d_a - lo_v, w_a)
            scat3(pos_b, s_b, d_b - lo_v, w_b)
            return cnt + (pc_a + pc_b)[0]

        return lax.fori_loop(0, QV2, filt, staged, unroll=10)

    def fire_g(b, s):
        pltpu.async_copy(table_h.at[fsrc2.at[b]], rowss[s], gsems[s])

    def wait_g(s):
        pltpu.make_async_copy(table_h.at[fsrc2.at[0]], rowss[s],
                              gsems[s]).wait()

    def fma_batch(b, s):
        base = b * G
        rows = rowss[s]

        def edge_add(e, _):
            dvec = fdst[pl.ds(base + e, 16)]
            wvec = fw[pl.ds(base + e, 16)]
            dr = dvec[0]
            wsp = jnp.full((16,), plsc.bitcast(wvec, _f32)[0], _f32)
            for j in range(8):
                sl = pl.ds(j * 16, 16)
                plsc.addupdate(acc.at[dr, sl], wsp * rows[e, sl])
            plsc.addupdate(dega.at[dr, pl.ds(0, 16)], onef)
            return 0

        def edge_max(e, _):
            dvec = fdst[pl.ds(base + e, 16)]
            wvec = fw[pl.ds(base + e, 16)]
            dr = dvec[0]
            wsp = jnp.full((16,), plsc.bitcast(wvec, _f32)[0], _f32)
            for j in range(8):
                sl = pl.ds(j * 16, 16)
                acc[dr, sl] = jnp.maximum(acc[dr, sl], wsp * rows[e, sl])
            return 0

        if mode == "add":
            lax.fori_loop(0, G, edge_add, 0, unroll=4)
        else:
            lax.fori_loop(0, G, edge_max, 0, unroll=4)

    def flush(staged):
        for mp in range(PADV):
            pos = jnp.full((16,), staged + mp * 16, _i32) + lane
            scat3(pos, zi, rangev, zi)

        nb = (staged + (G - 1)) // G
        fire_g(0, 0)

        def pairb(p, _):
            b0 = 2 * p
            wait_g(0)

            @pl.when(b0 + 1 < nb)
            def _():
                fire_g(b0 + 1, 1)

            fma_batch(b0, 0)

            @pl.when(b0 + 1 < nb)
            def _():
                wait_g(1)

                @pl.when(b0 + 2 < nb)
                def _():
                    fire_g(b0 + 2, 0)

                fma_batch(b0 + 1, 1)

            return 0

        lax.fori_loop(0, (nb + 1) // 2, pairb, 0)

    fire_e(0, 0)

    def pair(k2, staged):
        k = 2 * k2
        wait_e(0)
        fire_e(k + 1, 1)
        staged = filter_chunk(0, staged)
        do_f = staged >= MFLUSH

        @pl.when(do_f)
        def _():
            flush(staged)

        staged = jnp.where(do_f, 0, staged)

        wait_e(1)

        @pl.when(k + 2 < NQ)
        def _():
            fire_e(k + 2, 0)

        staged = filter_chunk(1, staged)
        do_f2 = staged >= MFLUSH

        @pl.when(do_f2)
        def _():
            flush(staged)

        return jnp.where(do_f2, 0, staged)

    staged = lax.fori_loop(0, NQ // 2, pair, jnp.int32(0))

    @pl.when(staged > 0)
    def _():
        flush(staged)

    pltpu.sync_copy(acc.at[pl.ds(0, RANGE), :], out_h.at[pl.ds(lo, RANGE), :])
    if mode == "add":
        pltpu.sync_copy(dega.at[pl.ds(0, RANGE), :],
                        deg_out_h.at[pl.ds(lo, RANGE), :])


def _sc_scratch(with_deg):
    out = [pltpu.VMEM((RANGE + 1, 128), _f32)]
    if with_deg:
        out.append(pltpu.VMEM((RANGE + 1, 16), _f32))
    out += [
        pltpu.VMEM((3, Q), _i32),
        pltpu.VMEM((3, Q), _i32),
        pltpu.VMEM((NROWS, G), _i32),
        pltpu.VMEM((MCAP,), _i32),
        pltpu.VMEM((MCAP,), _i32),
        pltpu.VMEM((G, 128), _f32),
        pltpu.VMEM((G, 128), _f32),
        pltpu.SemaphoreType.DMA,
        pltpu.SemaphoreType.DMA,
        pltpu.SemaphoreType.DMA,
        pltpu.SemaphoreType.DMA,
    ]
    return out


def _sc_mesh():
    return plsc.VectorSubcoreMesh(core_axis_name="c", subcore_axis_name="s")


@functools.partial(
    pl.kernel,
    out_type=[
        jax.ShapeDtypeStruct((NPAD, 128), _f32),
        jax.ShapeDtypeStruct((NPAD, 16), _f32),
        jax.ShapeDtypeStruct((NPAD, 128), _f32),
        jax.ShapeDtypeStruct((NPAD, 16), _f32),
    ],
    mesh=_sc_mesh(),
    scratch_types=_sc_scratch(True),
    compiler_params=pltpu.CompilerParams(needs_layout_passes=False),
)
def _sc_mean(x_h, ep0_h, ep1_h,
             s0_o, d0_o, s1_o, d1_o,
             acc, dega, eb0, eb1, fsrc2, fdst, fw, r0, r1,
             es0, es1, gs0, gs1):
    wid = lax.axis_index("s") * NC + lax.axis_index("c")
    ebufs = (eb0, eb1)
    esems = (es0, es1)
    _sc_rel(ep0_h, x_h, s0_o, d0_o, "add",
            acc, dega, ebufs, fsrc2, fdst, fw, (r0, r1), esems, (gs0, gs1), wid)
    _sc_rel(ep1_h, x_h, s1_o, d1_o, "add",
            acc, dega, ebufs, fsrc2, fdst, fw, (r0, r1), esems, (gs0, gs1), wid)


@functools.partial(
    pl.kernel,
    out_type=[
        jax.ShapeDtypeStruct((NPAD, 128), _f32),
        jax.ShapeDtypeStruct((NPAD, 128), _f32),
    ],
    mesh=_sc_mesh(),
    scratch_types=_sc_scratch(False),
    compiler_params=pltpu.CompilerParams(needs_layout_passes=False),
)
def _sc_pool(hp0_h, hp1_h, ep0_h, ep1_h,
             m0_o, m1_o,
             acc, eb0, eb1, fsrc2, fdst, fw, r0, r1,
             es0, es1, gs0, gs1):
    wid = lax.axis_index("s") * NC + lax.axis_index("c")
    ebufs = (eb0, eb1)
    esems = (es0, es1)
    _sc_rel(ep0_h, hp0_h, m0_o, None, "max",
            acc, None, ebufs, fsrc2, fdst, fw, (r0, r1), esems, (gs0, gs1), wid)
    _sc_rel(ep1_h, hp1_h, m1_o, None, "max",
            acc, None, ebufs, fsrc2, fdst, fw, (r0, r1), esems, (gs0, gs1), wid)


_BLK = 1000


def _tc1_body(x_ref, s0_ref, d0_ref, s1_ref, d1_ref,
              wsum_ref, w1n0_ref, w1n1_ref, b1_ref,
              wp0_ref, bp0_ref, wp1_ref, bp1_ref,
              w2ss_ref, b2_ref,
              hp0_ref, hp1_ref, oself_ref):
    deg0 = d0_ref[...][:, :1]
    deg1 = d1_ref[...][:, :1]
    agg0 = s0_ref[...] / jnp.maximum(deg0, 1.0)
    agg1 = s1_ref[...] / jnp.maximum(deg1, 1.0)
    x = x_ref[...]
    pre = (jnp.dot(x, wsum_ref[...], preferred_element_type=_f32)
           + jnp.dot(agg0, w1n0_ref[...], preferred_element_type=_f32)
           + jnp.dot(agg1, w1n1_ref[...], preferred_element_type=_f32)
           + b1_ref[...])
    h = jnp.maximum(pre, 0.0)
    hp0_ref[...] = jnp.maximum(
        jnp.dot(h, wp0_ref[...], preferred_element_type=_f32) + bp0_ref[...], 0.0)
    hp1_ref[...] = jnp.maximum(
        jnp.dot(h, wp1_ref[...], preferred_element_type=_f32) + bp1_ref[...], 0.0)
    oself_ref[...] = jnp.dot(h, w2ss_ref[...], preferred_element_type=_f32) + b2_ref[...]


def _tc2_body(oself_ref, m0_ref, m1_ref, w2n0_ref, w2n1_ref, out_ref):
    out_ref[...] = (oself_ref[...]
                    + jnp.dot(m0_ref[...], w2n0_ref[...], preferred_element_type=_f32)
                    + jnp.dot(m1_ref[...], w2n1_ref[...], preferred_element_type=_f32))


def _row_spec(cols):
    return pl.BlockSpec((_BLK, cols), lambda i: (i, 0))


def _full_spec(shape):
    nd = len(shape)
    return pl.BlockSpec(shape, lambda i: (0,) * nd)


def kernel(x, edge_index_r0, edge_index_r1, edge_weight_r0, edge_weight_r1,
           W1_self_r0, W1_neigh_r0, b1_r0, Wp_r0, bp_r0, W2_self_r0, W2_neigh_r0, b2_r0,
           W1_self_r1, W1_neigh_r1, b1_r1, Wp_r1, bp_r1, W2_self_r1, W2_neigh_r1, b2_r1):
    ep0 = jnp.concatenate(
        [edge_index_r0.astype(_i32),
         lax.bitcast_convert_type(edge_weight_r0, _i32)[None]], axis=0)
    ep1 = jnp.concatenate(
        [edge_index_r1.astype(_i32),
         lax.bitcast_convert_type(edge_weight_r1, _i32)[None]], axis=0)
    s0, d0, s1, d1 = _sc_mean(x, ep0, ep1)

    wsum = W1_self_r0 + W1_self_r1
    b1s = (b1_r0 + b1_r1).reshape(1, H)
    w2ss = W2_self_r0 + W2_self_r1
    b2s = (b2_r0 + b2_r1).reshape(1, C)

    hp0, hp1, oself = pl.pallas_call(
        _tc1_body,
        grid=(N // _BLK,),
        in_specs=[
            _row_spec(D), _row_spec(128), _row_spec(16),
            _row_spec(128), _row_spec(16),
            _full_spec((D, H)), _full_spec((D, H)), _full_spec((D, H)),
            _full_spec((1, H)),
            _full_spec((H, H)), _full_spec((1, H)),
            _full_spec((H, H)), _full_spec((1, H)),
            _full_spec((H, C)), _full_spec((1, C)),
        ],
        out_specs=[_row_spec(H), _row_spec(H), _row_spec(C)],
        out_shape=[
            jax.ShapeDtypeStruct((N, H), _f32),
            jax.ShapeDtypeStruct((N, H), _f32),
            jax.ShapeDtypeStruct((N, C), _f32),
        ],
    )(x, s0, d0, s1, d1, wsum, W1_neigh_r0, W1_neigh_r1, b1s,
      Wp_r0, bp_r0.reshape(1, H), Wp_r1, bp_r1.reshape(1, H), w2ss, b2s)

    m0, m1 = _sc_pool(hp0, hp1, ep0, ep1)

    logits = pl.pallas_call(
        _tc2_body,
        grid=(N // _BLK,),
        in_specs=[
            _row_spec(C), _row_spec(128), _row_spec(128),
            _full_spec((H, C)), _full_spec((H, C)),
        ],
        out_specs=_row_spec(C),
        out_shape=jax.ShapeDtypeStruct((N, C), _f32),
    )(oself, m0, m1, W2_neigh_r0, W2_neigh_r1)

    return logits


# speedup vs baseline: 1.0019x; 1.0019x over previous
import functools

import jax
import jax.numpy as jnp
from jax import lax
from jax.experimental import pallas as pl
from jax.experimental.pallas import tpu as pltpu
from jax.experimental.pallas import tpu_sc as plsc

N = 10000
E = 320000
D = 128
H = 128
C = 40

NC = 2
NS = 16
NW = NC * NS
NPAD = 10240
RANGE = NPAD // NW
Q = 640
NQ = E // Q
QV2 = Q // 32
G = 32
GSHIFT = 5
MFLUSH = 1280
MCAP = 2048
NROWS = MCAP // G
PADV = 3

_f32 = jnp.float32
_i32 = jnp.int32


def _sc_rel(ep_h, table_h, out_h, deg_out_h, mode,
            acc, dega, ebufs, fsrc2, fdst, fw, rowss, esems, gsems,
            wid):
    zf = jnp.zeros((16,), _f32)
    zi = jnp.zeros((16,), _i32)
    onef = jnp.ones((16,), _f32)
    rangev = jnp.full((16,), RANGE, _i32)

    lo = wid * RANGE
    lo_v = jnp.full((16,), lo, _i32)
    hi_v = jnp.full((16,), lo + RANGE, _i32)
    lane = lax.iota(_i32, 16)
    trash = jnp.full((16,), MCAP - 16, _i32) + lane
    one_i = jnp.ones((16,), _i32)
    gm1 = jnp.full((16,), G - 1, _i32)

    def zero_row(r, _):
        for j in range(8):
            acc[r, pl.ds(j * 16, 16)] = zf
        if mode == "add":
            dega[r, pl.ds(0, 16)] = zf
        return 0

    lax.fori_loop(0, RANGE + 1, zero_row, 0)

    def fire_e(k, s):
        pltpu.async_copy(ep_h.at[:, pl.ds(k * Q, Q)], ebufs[s], esems[s])

    def wait_e(s):
        pltpu.make_async_copy(ep_h.at[:, pl.ds(0, Q)], ebufs[s], esems[s]).wait()

    def scat3(pos, sv, dv, wv):
        plsc.store_scatter(fsrc2, [lax.shift_right_logical(pos, GSHIFT),
                                   pos & gm1], sv)
        plsc.store_scatter(fdst, [pos], dv)
        plsc.store_scatter(fw, [pos], wv)

    def filter_chunk(s, staged):
        eb = ebufs[s]

        def filt(i, cnt):
            off = i * 32
            cntv = jnp.full((16,), cnt, _i32)
            d_a = eb[1, pl.ds(off, 16)]
            s_a = eb[0, pl.ds(off, 16)]
            w_a = eb[2, pl.ds(off, 16)]
            d_b = eb[1, pl.ds(off + 16, 16)]
            s_b = eb[0, pl.ds(off + 16, 16)]
            w_b = eb[2, pl.ds(off + 16, 16)]
            m_a = (d_a >= lo_v) & (d_a < hi_v)
            m_b = (d_b >= lo_v) & (d_b < hi_v)
            inc_a = plsc.cumsum(m_a.astype(_i32))
            inc_b = plsc.cumsum(m_b.astype(_i32))
            pc_a = plsc.all_reduce_population_count(m_a)
            pc_b = plsc.all_reduce_population_count(m_b)
            pos_a = jnp.where(m_a, cntv + inc_a - one_i, trash)
            pos_b = jnp.where(m_b, cntv + pc_a + inc_b - one_i, trash)
            scat3(pos_a, s_a, d_a - lo_v, w_a)
            scat3(pos_b, s_b, d_b - lo_v, w_b)
            return cnt + (pc_a + pc_b)[0]

        return lax.fori_loop(0, QV2, filt, staged, unroll=5)

    def fire_g(b, s):
        pltpu.async_copy(table_h.at[fsrc2.at[b]], rowss[s], gsems[s])

    def wait_g(s):
        pltpu.make_async_copy(table_h.at[fsrc2.at[0]], rowss[s],
                              gsems[s]).wait()

    def fma_batch(b, s):
        base = b * G
        rows = rowss[s]

        def edge_add(e, _):
            dvec = fdst[pl.ds(base + e, 16)]
            wvec = fw[pl.ds(base + e, 16)]
            dr = dvec[0]
            wsp = jnp.full((16,), plsc.bitcast(wvec, _f32)[0], _f32)
            for j in range(8):
                sl = pl.ds(j * 16, 16)
                plsc.addupdate(acc.at[dr, sl], wsp * rows[e, sl])
            plsc.addupdate(dega.at[dr, pl.ds(0, 16)], onef)
            return 0

        def edge_max(e, _):
            dvec = fdst[pl.ds(base + e, 16)]
            wvec = fw[pl.ds(base + e, 16)]
            dr = dvec[0]
            wsp = jnp.full((16,), plsc.bitcast(wvec, _f32)[0], _f32)
            for j in range(8):
                sl = pl.ds(j * 16, 16)
                acc[dr, sl] = jnp.maximum(acc[dr, sl], wsp * rows[e, sl])
            return 0

        if mode == "add":
            lax.fori_loop(0, G, edge_add, 0, unroll=4)
        else:
            lax.fori_loop(0, G, edge_max, 0, unroll=4)

    def flush(staged):
        for mp in range(PADV):
            pos = jnp.full((16,), staged + mp * 16, _i32) + lane
            scat3(pos, zi, rangev, zi)

        nb = (staged + (G - 1)) // G
        fire_g(0, 0)

        def pairb(p, _):
            b0 = 2 * p
            wait_g(0)

            @pl.when(b0 + 1 < nb)
            def _():
                fire_g(b0 + 1, 1)

            fma_batch(b0, 0)

            @pl.when(b0 + 1 < nb)
            def _():
                wait_g(1)

                @pl.when(b0 + 2 < nb)
                def _():
                    fire_g(b0 + 2, 0)

                fma_batch(b0 + 1, 1)

            return 0

        lax.fori_loop(0, (nb + 1) // 2, pairb, 0)

    fire_e(0, 0)

    def pair(k2, staged):
        k = 2 * k2
        wait_e(0)
        fire_e(k + 1, 1)
        staged = filter_chunk(0, staged)
        do_f = staged >= MFLUSH

        @pl.when(do_f)
        def _():
            flush(staged)

        staged = jnp.where(do_f, 0, staged)

        wait_e(1)

        @pl.when(k + 2 < NQ)
        def _():
            fire_e(k + 2, 0)

        staged = filter_chunk(1, staged)
        do_f2 = staged >= MFLUSH

        @pl.when(do_f2)
        def _():
            flush(staged)

        return jnp.where(do_f2, 0, staged)

    staged = lax.fori_loop(0, NQ // 2, pair, jnp.int32(0))

    @pl.when(staged > 0)
    def _():
        flush(staged)

    pltpu.sync_copy(acc.at[pl.ds(0, RANGE), :], out_h.at[pl.ds(lo, RANGE), :])
    if mode == "add":
        pltpu.sync_copy(dega.at[pl.ds(0, RANGE), :],
                        deg_out_h.at[pl.ds(lo, RANGE), :])


def _sc_scratch(with_deg):
    out = [pltpu.VMEM((RANGE + 1, 128), _f32)]
    if with_deg:
        out.append(pltpu.VMEM((RANGE + 1, 16), _f32))
    out += [
        pltpu.VMEM((3, Q), _i32),
        pltpu.VMEM((3, Q), _i32),
        pltpu.VMEM((NROWS, G), _i32),
        pltpu.VMEM((MCAP,), _i32),
        pltpu.VMEM((MCAP,), _i32),
        pltpu.VMEM((G, 128), _f32),
        pltpu.VMEM((G, 128), _f32),
        pltpu.SemaphoreType.DMA,
        pltpu.SemaphoreType.DMA,
        pltpu.SemaphoreType.DMA,
        pltpu.SemaphoreType.DMA,
    ]
    return out


def _sc_mesh():
    return plsc.VectorSubcoreMesh(core_axis_name="c", subcore_axis_name="s")


@functools.partial(
    pl.kernel,
    out_type=[
        jax.ShapeDtypeStruct((NPAD, 128), _f32),
        jax.ShapeDtypeStruct((NPAD, 16), _f32),
        jax.ShapeDtypeStruct((NPAD, 128), _f32),
        jax.ShapeDtypeStruct((NPAD, 16), _f32),
    ],
    mesh=_sc_mesh(),
    scratch_types=_sc_scratch(True),
    compiler_params=pltpu.CompilerParams(needs_layout_passes=False),
)
def _sc_mean(x_h, ep0_h, ep1_h,
             s0_o, d0_o, s1_o, d1_o,
             acc, dega, eb0, eb1, fsrc2, fdst, fw, r0, r1,
             es0, es1, gs0, gs1):
    wid = lax.axis_index("s") * NC + lax.axis_index("c")
    ebufs = (eb0, eb1)
    esems = (es0, es1)
    _sc_rel(ep0_h, x_h, s0_o, d0_o, "add",
            acc, dega, ebufs, fsrc2, fdst, fw, (r0, r1), esems, (gs0, gs1), wid)
    _sc_rel(ep1_h, x_h, s1_o, d1_o, "add",
            acc, dega, ebufs, fsrc2, fdst, fw, (r0, r1), esems, (gs0, gs1), wid)


@functools.partial(
    pl.kernel,
    out_type=[
        jax.ShapeDtypeStruct((NPAD, 128), _f32),
        jax.ShapeDtypeStruct((NPAD, 128), _f32),
    ],
    mesh=_sc_mesh(),
    scratch_types=_sc_scratch(False),
    compiler_params=pltpu.CompilerParams(needs_layout_passes=False),
)
def _sc_pool(hp0_h, hp1_h, ep0_h, ep1_h,
             m0_o, m1_o,
             acc, eb0, eb1, fsrc2, fdst, fw, r0, r1,
             es0, es1, gs0, gs1):
    wid = lax.axis_index("s") * NC + lax.axis_index("c")
    ebufs = (eb0, eb1)
    esems = (es0, es1)
    _sc_rel(ep0_h, hp0_h, m0_o, None, "max",
            acc, None, ebufs, fsrc2, fdst, fw, (r0, r1), esems, (gs0, gs1), wid)
    _sc_rel(ep1_h, hp1_h, m1_o, None, "max",
            acc, None, ebufs, fsrc2, fdst, fw, (r0, r1), esems, (gs0, gs1), wid)


_BLK = 1000


def _tc1_body(x_ref, s0_ref, d0_ref, s1_ref, d1_ref,
              wsum_ref, w1n0_ref, w1n1_ref, b1_ref,
              wp0_ref, bp0_ref, wp1_ref, bp1_ref,
              w2ss_ref, b2_ref,
              hp0_ref, hp1_ref, oself_ref):
    deg0 = d0_ref[...][:, :1]
    deg1 = d1_ref[...][:, :1]
    agg0 = s0_ref[...] / jnp.maximum(deg0, 1.0)
    agg1 = s1_ref[...] / jnp.maximum(deg1, 1.0)
    x = x_ref[...]
    pre = (jnp.dot(x, wsum_ref[...], preferred_element_type=_f32)
           + jnp.dot(agg0, w1n0_ref[...], preferred_element_type=_f32)
           + jnp.dot(agg1, w1n1_ref[...], preferred_element_type=_f32)
           + b1_ref[...])
    h = jnp.maximum(pre, 0.0)
    hp0_ref[...] = jnp.maximum(
        jnp.dot(h, wp0_ref[...], preferred_element_type=_f32) + bp0_ref[...], 0.0)
    hp1_ref[...] = jnp.maximum(
        jnp.dot(h, wp1_ref[...], preferred_element_type=_f32) + bp1_ref[...], 0.0)
    oself_ref[...] = jnp.dot(h, w2ss_ref[...], preferred_element_type=_f32) + b2_ref[...]


def _tc2_body(oself_ref, m0_ref, m1_ref, w2n0_ref, w2n1_ref, out_ref):
    out_ref[...] = (oself_ref[...]
                    + jnp.dot(m0_ref[...], w2n0_ref[...], preferred_element_type=_f32)
                    + jnp.dot(m1_ref[...], w2n1_ref[...], preferred_element_type=_f32))


def _row_spec(cols):
    return pl.BlockSpec((_BLK, cols), lambda i: (i, 0))


def _full_spec(shape):
    nd = len(shape)
    return pl.BlockSpec(shape, lambda i: (0,) * nd)


def kernel(x, edge_index_r0, edge_index_r1, edge_weight_r0, edge_weight_r1,
           W1_self_r0, W1_neigh_r0, b1_r0, Wp_r0, bp_r0, W2_self_r0, W2_neigh_r0, b2_r0,
           W1_self_r1, W1_neigh_r1, b1_r1, Wp_r1, bp_r1, W2_self_r1, W2_neigh_r1, b2_r1):
    ep0 = jnp.concatenate(
        [edge_index_r0.astype(_i32),
         lax.bitcast_convert_type(edge_weight_r0, _i32)[None]], axis=0)
    ep1 = jnp.concatenate(
        [edge_index_r1.astype(_i32),
         lax.bitcast_convert_type(edge_weight_r1, _i32)[None]], axis=0)
    s0, d0, s1, d1 = _sc_mean(x, ep0, ep1)

    wsum = W1_self_r0 + W1_self_r1
    b1s = (b1_r0 + b1_r1).reshape(1, H)
    w2ss = W2_self_r0 + W2_self_r1
    b2s = (b2_r0 + b2_r1).reshape(1, C)

    hp0, hp1, oself = pl.pallas_call(
        _tc1_body,
        grid=(N // _BLK,),
        in_specs=[
            _row_spec(D), _row_spec(128), _row_spec(16),
            _row_spec(128), _row_spec(16),
            _full_spec((D, H)), _full_spec((D, H)), _full_spec((D, H)),
            _full_spec((1, H)),
            _full_spec((H, H)), _full_spec((1, H)),
            _full_spec((H, H)), _full_spec((1, H)),
            _full_spec((H, C)), _full_spec((1, C)),
        ],
        out_specs=[_row_spec(H), _row_spec(H), _row_spec(C)],
        out_shape=[
            jax.ShapeDtypeStruct((N, H), _f32),
            jax.ShapeDtypeStruct((N, H), _f32),
            jax.ShapeDtypeStruct((N, C), _f32),
        ],
    )(x, s0, d0, s1, d1, wsum, W1_neigh_r0, W1_neigh_r1, b1s,
      Wp_r0, bp_r0.reshape(1, H), Wp_r1, bp_r1.reshape(1, H), w2ss, b2s)

    m0, m1 = _sc_pool(hp0, hp1, ep0, ep1)

    logits = pl.pallas_call(
        _tc2_body,
        grid=(N // _BLK,),
        in_specs=[
            _row_spec(C), _row_spec(128), _row_spec(128),
            _full_spec((H, C)), _full_spec((H, C)),
        ],
        out_specs=_row_spec(C),
        out_shape=jax.ShapeDtypeStruct((N, C), _f32),
    )(oself, m0, m1, W2_neigh_r0, W2_neigh_r1)

    return logits
